# Initial kernel scaffold; baseline (speedup 1.0000x reference)
import jax
import jax.numpy as jnp
from jax.experimental import pallas as pl


def kernel(x, edge_weight, lin0_w, lin0_b, lin1_w, lin1_b, conv_w1, conv_w2, conv_w3, conv_w4, edge_index):
    raise NotImplementedError("write your pallas kernel here")


# speedup vs baseline: 2.1976x; 2.1976x over previous
import functools

import jax
import jax.numpy as jnp
import numpy as np
from jax import lax
from jax.experimental import pallas as pl
from jax.experimental.pallas import tpu as pltpu
from jax.experimental.pallas import tpu_sc as plsc

ALPHA = 0.1
THETA = 0.5
N_NODES = 50000
N_EDGES = 800000
HIDDEN = 64
IN_DIM = 33
OUT_DIM = 3

NC = 2
NS = 16
HALF = 25088
SCRAP = HALF
ACC_ROWS = HALF + 128
ROWS_PER_TILE = HALF // NS
CH = 128
CHUNKS = 16
EB = CH * CHUNKS
BLOCKS = 25
EPAD = NS * BLOCKS * EB
EROWS = EPAD // CH
LAST_ROWS = N_NODES - (HALF + (NS - 1) * ROWS_PER_TILE)


def _lane_splat(vec, lane):
    idx = jnp.full((16,), lane, jnp.int32)
    return jnp.take_along_axis(vec, idx, axis=0)


def _spmm_body(h_hbm, e_hbm, out_hbm, e_v, rows_v, acc_sh, gsem, ssem):
    c = lax.axis_index("c")
    s = lax.axis_index("s")
    lo = c * HALF
    zvec = jnp.zeros((16,), jnp.float32)

    def _zrow(r, carry):
        for k in range(HIDDEN // 16):
            rows_v[0, r, pl.ds(16 * k, 16)] = zvec
        return carry

    lax.fori_loop(0, CH, _zrow, 0)
    base = s * ROWS_PER_TILE
    for i in range(ROWS_PER_TILE // CH):
        pltpu.sync_copy(rows_v.at[0], acc_sh.at[pl.ds(base + i * CH, CH)])
    rem = ROWS_PER_TILE % CH
    if rem:
        pltpu.sync_copy(rows_v.at[0, pl.ds(0, rem)],
                        acc_sh.at[pl.ds(base + (ROWS_PER_TILE // CH) * CH, rem)])
    plsc.subcore_barrier()

    def _block(b, carry):
        row0 = (s * BLOCKS + b) * CHUNKS
        pltpu.sync_copy(e_hbm.at[pl.ds(row0, CHUNKS)], e_v)
        for j in range(CHUNKS):
            for k in range(CH // 16):
                d = e_v[j, 1, pl.ds(16 * k, 16)]
                l = d - lo
                oob = (l < 0) | (l >= HALF)
                e_v[j, 1, pl.ds(16 * k, 16)] = jnp.where(oob, SCRAP, l)

        def _scale(bi, j):
            def _grp(g, carry):
                ew16 = lax.bitcast_convert_type(
                    e_v[j, 2, pl.ds(16 * g, 16)], jnp.float32)
                for l in range(16):
                    ewl = _lane_splat(ew16, l)
                    r = 16 * g + l
                    for k in range(HIDDEN // 16):
                        rows_v[bi, r, pl.ds(16 * k, 16)] = (
                            rows_v[bi, r, pl.ds(16 * k, 16)] * ewl)
                return carry
            lax.fori_loop(0, CH // 16, _grp, 0)

        cps = [None] * CHUNKS
        g = pltpu.async_copy(h_hbm.at[e_v.at[0, 0]], rows_v.at[0], gsem[0])
        for j in range(CHUNKS):
            bi = j % 2
            if j + 1 < CHUNKS:
                if j - 1 >= 0:
                    cps[j - 1].wait()
                gn = pltpu.async_copy(h_hbm.at[e_v.at[j + 1, 0]],
                                      rows_v.at[(j + 1) % 2],
                                      gsem[(j + 1) % 2])
            g.wait()
            _scale(bi, j)
            cps[j] = pltpu.async_copy(rows_v.at[bi], acc_sh.at[e_v.at[j, 1]],
                                      ssem[bi], add=True)
            if j + 1 < CHUNKS:
                g = gn
        cps[CHUNKS - 2].wait()
        cps[CHUNKS - 1].wait()
        return carry

    lax.fori_loop(0, BLOCKS, _block, 0)
    plsc.subcore_barrier()

    glob = lo + base
    is_last = (c == NC - 1) & (s == NS - 1)

    @pl.when(jnp.logical_not(is_last))
    def _():
        pltpu.sync_copy(acc_sh.at[pl.ds(base, ROWS_PER_TILE)],
                        out_hbm.at[pl.ds(glob, ROWS_PER_TILE)])

    @pl.when(is_last)
    def _():
        pltpu.sync_copy(acc_sh.at[pl.ds(base, LAST_ROWS)],
                        out_hbm.at[pl.ds(glob, LAST_ROWS)])


_spmm = pl.kernel(
    _spmm_body,
    out_type=jax.ShapeDtypeStruct((N_NODES, HIDDEN), jnp.float32),
    mesh=plsc.VectorSubcoreMesh(core_axis_name="c", subcore_axis_name="s",
                                num_cores=NC, num_subcores=NS),
    compiler_params=pltpu.CompilerParams(use_tc_tiling_on_sc=False),
    scratch_types=[
        pltpu.VMEM((CHUNKS, 3, CH), jnp.int32),
        pltpu.VMEM((2, CH, HIDDEN), jnp.float32),
        pltpu.VMEM_SHARED((ACC_ROWS, HIDDEN), jnp.float32),
        [pltpu.SemaphoreType.DMA, pltpu.SemaphoreType.DMA],
        [pltpu.SemaphoreType.DMA, pltpu.SemaphoreType.DMA],
    ],
)


R_BLK = 2000


def _in_proj_body(x_ref, w_ref, b_ref, o_ref):
    y = jnp.dot(x_ref[...], w_ref[...], preferred_element_type=jnp.float32)
    o_ref[...] = jnp.maximum(y + b_ref[...], 0.0)


_in_proj = pl.pallas_call(
    _in_proj_body,
    grid=(N_NODES // R_BLK,),
    in_specs=[
        pl.BlockSpec((R_BLK, IN_DIM), lambda i: (i, 0)),
        pl.BlockSpec((IN_DIM, HIDDEN), lambda i: (0, 0)),
        pl.BlockSpec((1, HIDDEN), lambda i: (0, 0)),
    ],
    out_specs=pl.BlockSpec((R_BLK, HIDDEN), lambda i: (i, 0)),
    out_shape=jax.ShapeDtypeStruct((N_NODES, HIDDEN), jnp.float32),
)


def _combine_body(beta, agg_ref, x0_ref, h_ref, w_ref, o_ref):
    t = (1.0 - ALPHA) * agg_ref[...] + ALPHA * x0_ref[...]
    u = (1.0 - beta) * t + beta * jnp.dot(
        t, w_ref[...], preferred_element_type=jnp.float32)
    o_ref[...] = h_ref[...] + jnp.maximum(u, 0.0)


def _make_combine(beta):
    return pl.pallas_call(
        functools.partial(_combine_body, beta),
        grid=(N_NODES // R_BLK,),
        in_specs=[
            pl.BlockSpec((R_BLK, HIDDEN), lambda i: (i, 0)),
            pl.BlockSpec((R_BLK, HIDDEN), lambda i: (i, 0)),
            pl.BlockSpec((R_BLK, HIDDEN), lambda i: (i, 0)),
            pl.BlockSpec((HIDDEN, HIDDEN), lambda i: (0, 0)),
        ],
        out_specs=pl.BlockSpec((R_BLK, HIDDEN), lambda i: (i, 0)),
        out_shape=jax.ShapeDtypeStruct((N_NODES, HIDDEN), jnp.float32),
    )


def _out_proj_body(h_ref, w_ref, b_ref, o_ref):
    y = jnp.dot(h_ref[...], w_ref[...], preferred_element_type=jnp.float32)
    o_ref[...] = y + b_ref[...]


_out_proj = pl.pallas_call(
    _out_proj_body,
    grid=(N_NODES // R_BLK,),
    in_specs=[
        pl.BlockSpec((R_BLK, HIDDEN), lambda i: (i, 0)),
        pl.BlockSpec((HIDDEN, OUT_DIM), lambda i: (0, 0)),
        pl.BlockSpec((1, OUT_DIM), lambda i: (0, 0)),
    ],
    out_specs=pl.BlockSpec((R_BLK, OUT_DIM), lambda i: (i, 0)),
    out_shape=jax.ShapeDtypeStruct((N_NODES, OUT_DIM), jnp.float32),
)


def kernel(x, edge_weight, lin0_w, lin0_b, lin1_w, lin1_b,
           conv_w1, conv_w2, conv_w3, conv_w4, edge_index):
    pad = EPAD - N_EDGES
    srcp = jnp.concatenate(
        [edge_index[0], jnp.zeros((pad,), jnp.int32)]).reshape(EROWS, CH)
    dstp = jnp.concatenate(
        [edge_index[1],
         jnp.full((pad,), NC * HALF + 7, jnp.int32)]).reshape(EROWS, CH)
    ewp = jnp.concatenate(
        [edge_weight, jnp.zeros((pad,), jnp.float32)]).reshape(EROWS, CH)
    edges = jnp.stack(
        [srcp, dstp, lax.bitcast_convert_type(ewp, jnp.int32)], axis=1)

    h = _in_proj(x, lin0_w.T, lin0_b.reshape(1, HIDDEN))
    x0 = h
    for i, w in enumerate([conv_w1, conv_w2, conv_w3, conv_w4]):
        beta = float(np.log(THETA / (i + 1) + 1.0))
        agg = _spmm(h, edges)
        h = _make_combine(beta)(agg, x0, h, w)
    return _out_proj(h, lin1_w.T, lin1_b.reshape(1, OUT_DIM))
